# baseline (speedup 1.0000x reference)
import functools

import jax
import jax.numpy as jnp
from jax.experimental import pallas as pl
from jax.experimental.pallas import tpu as pltpu


_NEG = -1e30


def _router_body(x_ref, wg_ref, mask_ref, coef_ref, acc_ref, xn_ref,
                 running_ref, *, cap, n_exp):
    i = pl.program_id(0)

    @pl.when(i == 0)
    def _():
        running_ref[...] = jnp.zeros_like(running_ref)

    x = x_ref[...]
    t = x.shape[0]
    logits = jnp.dot(x, wg_ref[...], preferred_element_type=jnp.float32)
    lane = jax.lax.broadcasted_iota(jnp.int32, logits.shape, 1)
    logits = jnp.where(lane < n_exp - 1, logits, _NEG)
    m = jnp.max(logits, axis=1, keepdims=True)
    z = jnp.exp(logits - m)
    p = z / jnp.sum(z, axis=1, keepdims=True)

    e1 = jnp.argmax(p, axis=1).reshape(t, 1)
    oh1 = (lane == e1)
    p1 = jnp.max(p, axis=1, keepdims=True)
    pm = jnp.where(oh1, -1.0, p)
    e2 = jnp.argmax(pm, axis=1).reshape(t, 1)
    oh2 = (lane == e2)
    p2 = jnp.max(pm, axis=1, keepdims=True)
    wsum = p1 + p2
    w1 = p1 / wsum
    w2 = p2 / wsum

    oh1f = oh1.astype(jnp.float32)
    oh2f = oh2.astype(jnp.float32)
    s = oh1f + oh2f
    row = jax.lax.broadcasted_iota(jnp.int32, (t, t), 0)
    col = jax.lax.broadcasted_iota(jnp.int32, (t, t), 1)
    lstrict = (row > col).astype(jnp.bfloat16)
    cum = jnp.dot(lstrict, s.astype(jnp.bfloat16),
                  preferred_element_type=jnp.float32)
    base = running_ref[0:1, 0:s.shape[1]]
    cumx = cum + base
    rank1 = jnp.sum(oh1f * cumx, axis=1, keepdims=True)
    rank2 = jnp.sum(oh2f * cumx, axis=1, keepdims=True)
    sv1 = (rank1 < cap).astype(jnp.float32)
    sv2 = (rank2 < cap).astype(jnp.float32)
    wm1 = w1 * sv1
    wm2 = w2 * sv2

    mf = mask_ref[...]
    coef_e = oh1f * wm1 + oh2f * wm2
    third = jnp.float32(1.0 / 3.0)
    coef_ref[...] = jnp.concatenate(
        [mf * third, (2.0 * third) * mf * coef_e[:, : n_exp - 1]], axis=1)
    c_res = mf * (third + (2.0 * third) * (wm1 + wm2))
    acc_ref[...] = c_res * x

    mu = jnp.mean(x, axis=1, keepdims=True)
    var = jnp.mean((x - mu) ** 2, axis=1, keepdims=True)
    xn = (x - mu) / jnp.sqrt(var + 1e-6)
    xn_ref[...] = xn.astype(jnp.bfloat16)

    running_ref[0:1, 0:s.shape[1]] = base + jnp.sum(s, axis=0, keepdims=True)


def _ffn_body(coef_ref, xn_ref, cond_ref, acc_ref, wada_ref, bada_ref,
              w1_ref, w2_ref, out_ref, *, ffn_chunks, e):
    cond = cond_ref[...]
    t = cond.shape[0]
    ada = jnp.dot(cond, wada_ref[0], preferred_element_type=jnp.float32)
    ada = ada + bada_ref[0]
    d = ada.shape[1] // 3
    shift = ada[:, :d]
    scale = ada[:, d:2 * d]
    gate = ada[:, 2 * d:]
    h = xn_ref[...].astype(jnp.float32) * (1.0 + scale) + shift
    hb = h.astype(jnp.bfloat16)

    w1 = w1_ref[0]
    w2 = w2_ref[0]
    ff = w1.shape[1]
    ck = ff // ffn_chunks
    o = jnp.zeros((t, d), jnp.float32)
    for c in range(ffn_chunks):
        f = jnp.dot(hb, w1[:, c * ck:(c + 1) * ck],
                    preferred_element_type=jnp.float32)
        fb = jax.nn.gelu(f).astype(jnp.bfloat16)
        o = o + jnp.dot(fb, w2[c * ck:(c + 1) * ck, :],
                        preferred_element_type=jnp.float32)

    cw = coef_ref[:, e:e + 1]
    out_ref[...] = acc_ref[...] + cw * (gate * o)


def kernel(x, cond, mask, W_gate, Wada_sh, bada_sh, W1_sh, W2_sh,
           Wada_e, bada_e, W1_e, W2_e):
    b, n, d = x.shape
    nt = b * n
    dc = cond.shape[-1]
    n_exp = Wada_e.shape[0]
    cap = int(1.25 * 2 * nt / n_exp)

    xf = x.reshape(nt, d)
    cf = cond.reshape(nt, dc).astype(jnp.bfloat16)
    mf = mask.reshape(nt, 1)
    wg = jnp.pad(W_gate, ((0, 0), (0, 1)))

    wada_all = jnp.concatenate(
        [Wada_sh[None], Wada_e[: n_exp - 1]], axis=0).astype(jnp.bfloat16)
    bada_all = jnp.concatenate(
        [bada_sh[None], bada_e[: n_exp - 1]], axis=0).reshape(n_exp, 1, 3 * d)
    w1_all = jnp.concatenate(
        [W1_sh[None], W1_e[: n_exp - 1]], axis=0).astype(jnp.bfloat16)
    w2_all = jnp.concatenate(
        [W2_sh[None], W2_e[: n_exp - 1]], axis=0).astype(jnp.bfloat16)
    ff = w1_all.shape[2]

    rt = 512
    coefw, acc0, xnb = pl.pallas_call(
        functools.partial(_router_body, cap=cap, n_exp=n_exp),
        grid=(nt // rt,),
        in_specs=[
            pl.BlockSpec((rt, d), lambda i: (i, 0)),
            pl.BlockSpec((d, n_exp), lambda i: (0, 0)),
            pl.BlockSpec((rt, 1), lambda i: (i, 0)),
        ],
        out_specs=[
            pl.BlockSpec((rt, n_exp), lambda i: (i, 0)),
            pl.BlockSpec((rt, d), lambda i: (i, 0)),
            pl.BlockSpec((rt, d), lambda i: (i, 0)),
        ],
        out_shape=[
            jax.ShapeDtypeStruct((nt, n_exp), jnp.float32),
            jax.ShapeDtypeStruct((nt, d), jnp.float32),
            jax.ShapeDtypeStruct((nt, d), jnp.bfloat16),
        ],
        scratch_shapes=[pltpu.VMEM((8, 128), jnp.float32)],
    )(xf, wg, mf)

    ft = 256
    acc = acc0
    for e in range(n_exp):
        acc = pl.pallas_call(
            functools.partial(_ffn_body, ffn_chunks=4, e=e),
            grid=(nt // ft,),
            in_specs=[
                pl.BlockSpec((ft, n_exp), lambda t: (t, 0)),
                pl.BlockSpec((ft, d), lambda t: (t, 0)),
                pl.BlockSpec((ft, dc), lambda t: (t, 0)),
                pl.BlockSpec((ft, d), lambda t: (t, 0)),
                pl.BlockSpec((1, dc, 3 * d), lambda t, e=e: (e, 0, 0)),
                pl.BlockSpec((1, 1, 3 * d), lambda t, e=e: (e, 0, 0)),
                pl.BlockSpec((1, d, ff), lambda t, e=e: (e, 0, 0)),
                pl.BlockSpec((1, ff, d), lambda t, e=e: (e, 0, 0)),
            ],
            out_specs=pl.BlockSpec((ft, d), lambda t: (t, 0)),
            out_shape=jax.ShapeDtypeStruct((nt, d), jnp.float32),
            input_output_aliases={3: 0},
        )(coefw, xnb, cf, acc, wada_all, bada_all, w1_all, w2_all)

    return acc.reshape(b, n, d)


# speedup vs baseline: 1.2865x; 1.2865x over previous
import functools

import jax
import jax.numpy as jnp
from jax.experimental import pallas as pl
from jax.experimental.pallas import tpu as pltpu


_NEG = -1e30


def _router_body(x_ref, wg_ref, mask_ref, acc_ref, xn_ref,
                 s0_ref, w0_ref, s1_ref, w1_ref, running_ref,
                 *, cap, n_exp):
    i = pl.program_id(0)

    @pl.when(i == 0)
    def _():
        running_ref[...] = jnp.zeros_like(running_ref)

    x = x_ref[...]
    t = x.shape[0]
    logits = jnp.dot(x, wg_ref[...], preferred_element_type=jnp.float32)
    lane = jax.lax.broadcasted_iota(jnp.int32, logits.shape, 1)
    logits = jnp.where(lane < n_exp - 1, logits, _NEG)
    m = jnp.max(logits, axis=1, keepdims=True)
    z = jnp.exp(logits - m)
    p = z / jnp.sum(z, axis=1, keepdims=True)

    e1 = jnp.argmax(p, axis=1).reshape(t, 1)
    oh1 = (lane == e1)
    p1 = jnp.max(p, axis=1, keepdims=True)
    pm = jnp.where(oh1, -1.0, p)
    e2 = jnp.argmax(pm, axis=1).reshape(t, 1)
    oh2 = (lane == e2)
    p2 = jnp.max(pm, axis=1, keepdims=True)
    wsum = p1 + p2
    wa = p1 / wsum
    wb = p2 / wsum

    oh1f = oh1.astype(jnp.float32)
    oh2f = oh2.astype(jnp.float32)
    s = oh1f + oh2f
    row = jax.lax.broadcasted_iota(jnp.int32, (t, t), 0)
    col = jax.lax.broadcasted_iota(jnp.int32, (t, t), 1)
    lstrict = (row > col).astype(jnp.bfloat16)
    cum = jnp.dot(lstrict, s.astype(jnp.bfloat16),
                  preferred_element_type=jnp.float32)
    base = running_ref[0:1, 0:s.shape[1]]
    cumx = cum + base
    rank1 = jnp.sum(oh1f * cumx, axis=1, keepdims=True)
    rank2 = jnp.sum(oh2f * cumx, axis=1, keepdims=True)
    sv1 = rank1 < cap
    sv2 = rank2 < cap

    mf = mask_ref[...]
    third = jnp.float32(1.0 / 3.0)
    wm1 = jnp.where(sv1, wa, 0.0)
    wm2 = jnp.where(sv2, wb, 0.0)
    c_res = mf * (third + (2.0 * third) * (wm1 + wm2))
    acc_ref[...] = c_res * x

    s0_ref[...] = jnp.where(sv1, e1 * cap + rank1.astype(jnp.int32), -1)
    s1_ref[...] = jnp.where(sv2, e2 * cap + rank2.astype(jnp.int32), -1)
    w0_ref[...] = (2.0 * third) * mf * wm1
    w1_ref[...] = (2.0 * third) * mf * wm2

    mu = jnp.mean(x, axis=1, keepdims=True)
    var = jnp.mean((x - mu) ** 2, axis=1, keepdims=True)
    xn = (x - mu) / jnp.sqrt(var + 1e-6)
    xn_ref[...] = xn.astype(jnp.bfloat16)

    running_ref[0:1, 0:s.shape[1]] = base + jnp.sum(s, axis=0, keepdims=True)


def _gather_body(s0_ref, s1_ref, xn_ref, cond_ref, gxn_ref, gcond_ref,
                 *, blk):
    i = pl.program_id(0)
    n = s0_ref.shape[1]
    srow = (jax.lax.broadcasted_iota(jnp.int32, (blk, n), 0).astype(jnp.float32)
            + jnp.float32(i * blk))
    s0b = jnp.broadcast_to(s0_ref[...].astype(jnp.float32), (blk, n))
    s1b = jnp.broadcast_to(s1_ref[...].astype(jnp.float32), (blk, n))
    p = (jnp.maximum(1.0 - jnp.abs(s0b - srow), 0.0)
         + jnp.maximum(1.0 - jnp.abs(s1b - srow), 0.0)).astype(jnp.bfloat16)
    gxn_ref[...] = jnp.dot(
        p, xn_ref[...], preferred_element_type=jnp.float32
    ).astype(jnp.bfloat16)
    gcond_ref[...] = jnp.dot(
        p, cond_ref[...], preferred_element_type=jnp.float32
    ).astype(jnp.bfloat16)


def _gmlp(xn, cond, wada_ref, bada_ref, w1_ref, w2_ref, ffn_chunks):
    t = cond.shape[0]
    ada = jnp.dot(cond, wada_ref[0], preferred_element_type=jnp.float32)
    ada = ada + bada_ref[0]
    d = ada.shape[1] // 3
    shift = ada[:, :d]
    scale = ada[:, d:2 * d]
    gate = ada[:, 2 * d:]
    h = xn.astype(jnp.float32) * (1.0 + scale) + shift
    hb = h.astype(jnp.bfloat16)
    w1 = w1_ref[0]
    w2 = w2_ref[0]
    ck = w1.shape[1] // ffn_chunks
    o = jnp.zeros((t, d), jnp.float32)
    for c in range(ffn_chunks):
        f = jnp.dot(hb, w1[:, c * ck:(c + 1) * ck],
                    preferred_element_type=jnp.float32)
        fb = jax.nn.gelu(f).astype(jnp.bfloat16)
        o = o + jnp.dot(fb, w2[c * ck:(c + 1) * ck, :],
                        preferred_element_type=jnp.float32)
    return gate * o


def _expert_ffn_body(gxn_ref, gcond_ref, wada_ref, bada_ref, w1_ref, w2_ref,
                     eo_ref, *, ffn_chunks):
    gh = _gmlp(gxn_ref[...], gcond_ref[...], wada_ref, bada_ref,
               w1_ref, w2_ref, ffn_chunks)
    eo_ref[...] = gh.astype(jnp.bfloat16)


def _shared_ffn_body(xn_ref, cond_ref, mask_ref, acc_ref, wada_ref, bada_ref,
                     w1_ref, w2_ref, out_ref, *, ffn_chunks):
    gh = _gmlp(xn_ref[...], cond_ref[...], wada_ref, bada_ref,
               w1_ref, w2_ref, ffn_chunks)
    out_ref[...] = acc_ref[...] + (jnp.float32(1.0 / 3.0) * mask_ref[...]) * gh


def _scatter_body(s0_ref, w0_ref, s1_ref, w1_ref, eo_ref, acc_ref, out_ref,
                  *, n_slots):
    t = s0_ref.shape[0]
    siota = jax.lax.broadcasted_iota(jnp.int32, (t, n_slots), 1).astype(
        jnp.float32)
    hit0 = jnp.maximum(1.0 - jnp.abs(s0_ref[...].astype(jnp.float32) - siota),
                       0.0)
    hit1 = jnp.maximum(1.0 - jnp.abs(s1_ref[...].astype(jnp.float32) - siota),
                       0.0)
    pw = (hit0 * w0_ref[...] + hit1 * w1_ref[...]).astype(jnp.bfloat16)
    moe = jnp.dot(pw, eo_ref[...], preferred_element_type=jnp.float32)
    out_ref[...] = acc_ref[...] + moe


def kernel(x, cond, mask, W_gate, Wada_sh, bada_sh, W1_sh, W2_sh,
           Wada_e, bada_e, W1_e, W2_e):
    b, n, d = x.shape
    nt = b * n
    dc = cond.shape[-1]
    n_exp = Wada_e.shape[0]
    cap = int(1.25 * 2 * nt / n_exp)
    n_slots = (n_exp - 1) * cap

    xf = x.reshape(nt, d)
    cf = cond.reshape(nt, dc).astype(jnp.bfloat16)
    mf = mask.reshape(nt, 1)
    wg = jnp.pad(W_gate, ((0, 0), (0, 1)))

    wada_all = jnp.concatenate(
        [Wada_sh[None], Wada_e[: n_exp - 1]], axis=0).astype(jnp.bfloat16)
    bada_all = jnp.concatenate(
        [bada_sh[None], bada_e[: n_exp - 1]], axis=0).reshape(n_exp, 1, 3 * d)
    w1_all = jnp.concatenate(
        [W1_sh[None], W1_e[: n_exp - 1]], axis=0).astype(jnp.bfloat16)
    w2_all = jnp.concatenate(
        [W2_sh[None], W2_e[: n_exp - 1]], axis=0).astype(jnp.bfloat16)
    ff = w1_all.shape[2]

    rt = 512
    acc0, xnb, s0, w0, s1, w1 = pl.pallas_call(
        functools.partial(_router_body, cap=cap, n_exp=n_exp),
        grid=(nt // rt,),
        in_specs=[
            pl.BlockSpec((rt, d), lambda i: (i, 0)),
            pl.BlockSpec((d, n_exp), lambda i: (0, 0)),
            pl.BlockSpec((rt, 1), lambda i: (i, 0)),
        ],
        out_specs=[
            pl.BlockSpec((rt, d), lambda i: (i, 0)),
            pl.BlockSpec((rt, d), lambda i: (i, 0)),
            pl.BlockSpec((rt, 1), lambda i: (i, 0)),
            pl.BlockSpec((rt, 1), lambda i: (i, 0)),
            pl.BlockSpec((rt, 1), lambda i: (i, 0)),
            pl.BlockSpec((rt, 1), lambda i: (i, 0)),
        ],
        out_shape=[
            jax.ShapeDtypeStruct((nt, d), jnp.float32),
            jax.ShapeDtypeStruct((nt, d), jnp.bfloat16),
            jax.ShapeDtypeStruct((nt, 1), jnp.int32),
            jax.ShapeDtypeStruct((nt, 1), jnp.float32),
            jax.ShapeDtypeStruct((nt, 1), jnp.int32),
            jax.ShapeDtypeStruct((nt, 1), jnp.float32),
        ],
        scratch_shapes=[pltpu.VMEM((8, 128), jnp.float32)],
    )(xf, wg, mf)

    s0r = s0.reshape(1, nt)
    s1r = s1.reshape(1, nt)

    gb = 256
    gxn, gcond = pl.pallas_call(
        functools.partial(_gather_body, blk=gb),
        grid=(n_slots // gb,),
        in_specs=[
            pl.BlockSpec((1, nt), lambda i: (0, 0)),
            pl.BlockSpec((1, nt), lambda i: (0, 0)),
            pl.BlockSpec((nt, d), lambda i: (0, 0)),
            pl.BlockSpec((nt, dc), lambda i: (0, 0)),
        ],
        out_specs=[
            pl.BlockSpec((gb, d), lambda i: (i, 0)),
            pl.BlockSpec((gb, dc), lambda i: (i, 0)),
        ],
        out_shape=[
            jax.ShapeDtypeStruct((n_slots, d), jnp.bfloat16),
            jax.ShapeDtypeStruct((n_slots, dc), jnp.bfloat16),
        ],
    )(s0r, s1r, xnb, cf)

    ft = 256
    spe = cap // ft
    eo = pl.pallas_call(
        functools.partial(_expert_ffn_body, ffn_chunks=4),
        grid=(n_exp - 1, spe),
        in_specs=[
            pl.BlockSpec((ft, d), lambda e, c: (e * spe + c, 0)),
            pl.BlockSpec((ft, dc), lambda e, c: (e * spe + c, 0)),
            pl.BlockSpec((1, dc, 3 * d), lambda e, c: (e + 1, 0, 0)),
            pl.BlockSpec((1, 1, 3 * d), lambda e, c: (e + 1, 0, 0)),
            pl.BlockSpec((1, d, ff), lambda e, c: (e + 1, 0, 0)),
            pl.BlockSpec((1, ff, d), lambda e, c: (e + 1, 0, 0)),
        ],
        out_specs=pl.BlockSpec((ft, d), lambda e, c: (e * spe + c, 0)),
        out_shape=jax.ShapeDtypeStruct((n_slots, d), jnp.bfloat16),
    )(gxn, gcond, wada_all, bada_all, w1_all, w2_all)

    acc1 = pl.pallas_call(
        functools.partial(_shared_ffn_body, ffn_chunks=4),
        grid=(nt // ft,),
        in_specs=[
            pl.BlockSpec((ft, d), lambda t: (t, 0)),
            pl.BlockSpec((ft, dc), lambda t: (t, 0)),
            pl.BlockSpec((ft, 1), lambda t: (t, 0)),
            pl.BlockSpec((ft, d), lambda t: (t, 0)),
            pl.BlockSpec((1, dc, 3 * d), lambda t: (0, 0, 0)),
            pl.BlockSpec((1, 1, 3 * d), lambda t: (0, 0, 0)),
            pl.BlockSpec((1, d, ff), lambda t: (0, 0, 0)),
            pl.BlockSpec((1, ff, d), lambda t: (0, 0, 0)),
        ],
        out_specs=pl.BlockSpec((ft, d), lambda t: (t, 0)),
        out_shape=jax.ShapeDtypeStruct((nt, d), jnp.float32),
        input_output_aliases={3: 0},
    )(xnb, cf, mf, acc0, wada_all, bada_all, w1_all, w2_all)

    out = pl.pallas_call(
        functools.partial(_scatter_body, n_slots=n_slots),
        grid=(nt // ft,),
        in_specs=[
            pl.BlockSpec((ft, 1), lambda t: (t, 0)),
            pl.BlockSpec((ft, 1), lambda t: (t, 0)),
            pl.BlockSpec((ft, 1), lambda t: (t, 0)),
            pl.BlockSpec((ft, 1), lambda t: (t, 0)),
            pl.BlockSpec((n_slots, d), lambda t: (0, 0)),
            pl.BlockSpec((ft, d), lambda t: (t, 0)),
        ],
        out_specs=pl.BlockSpec((ft, d), lambda t: (t, 0)),
        out_shape=jax.ShapeDtypeStruct((nt, d), jnp.float32),
        input_output_aliases={5: 0},
    )(s0, w0, s1, w1, eo, acc1)

    return out.reshape(b, n, d)


# speedup vs baseline: 1.3210x; 1.0268x over previous
import functools

import jax
import jax.numpy as jnp
from jax.experimental import pallas as pl
from jax.experimental.pallas import tpu as pltpu


_NEG = -1e30


def _router_body(x_ref, wg_ref, mask_ref, acc_ref, xn_ref,
                 s0_ref, w0_ref, s1_ref, w1_ref, running_ref,
                 *, cap, n_exp):
    i = pl.program_id(0)

    @pl.when(i == 0)
    def _():
        running_ref[...] = jnp.zeros_like(running_ref)

    x = x_ref[...]
    t = x.shape[0]
    logits = jnp.dot(x, wg_ref[...], preferred_element_type=jnp.float32)
    lane = jax.lax.broadcasted_iota(jnp.int32, logits.shape, 1)
    logits = jnp.where(lane < n_exp - 1, logits, _NEG)
    m = jnp.max(logits, axis=1, keepdims=True)
    z = jnp.exp(logits - m)
    p = z / jnp.sum(z, axis=1, keepdims=True)

    e1 = jnp.argmax(p, axis=1).reshape(t, 1)
    oh1 = (lane == e1)
    p1 = jnp.max(p, axis=1, keepdims=True)
    pm = jnp.where(oh1, -1.0, p)
    e2 = jnp.argmax(pm, axis=1).reshape(t, 1)
    oh2 = (lane == e2)
    p2 = jnp.max(pm, axis=1, keepdims=True)
    wsum = p1 + p2
    wa = p1 / wsum
    wb = p2 / wsum

    oh1f = oh1.astype(jnp.float32)
    oh2f = oh2.astype(jnp.float32)
    s = oh1f + oh2f
    row = jax.lax.broadcasted_iota(jnp.int32, (t, t), 0)
    col = jax.lax.broadcasted_iota(jnp.int32, (t, t), 1)
    lstrict = (row > col).astype(jnp.bfloat16)
    cum = jnp.dot(lstrict, s.astype(jnp.bfloat16),
                  preferred_element_type=jnp.float32)
    base = running_ref[0:1, 0:s.shape[1]]
    cumx = cum + base
    rank1 = jnp.sum(oh1f * cumx, axis=1, keepdims=True)
    rank2 = jnp.sum(oh2f * cumx, axis=1, keepdims=True)
    sv1 = rank1 < cap
    sv2 = rank2 < cap

    mf = mask_ref[...]
    third = jnp.float32(1.0 / 3.0)
    wm1 = jnp.where(sv1, wa, 0.0)
    wm2 = jnp.where(sv2, wb, 0.0)
    c_res = mf * (third + (2.0 * third) * (wm1 + wm2))
    acc_ref[...] = c_res * x

    s0_ref[...] = jnp.where(sv1, e1 * cap + rank1.astype(jnp.int32), -1)
    s1_ref[...] = jnp.where(sv2, e2 * cap + rank2.astype(jnp.int32), -1)
    w0_ref[...] = (2.0 * third) * mf * wm1
    w1_ref[...] = (2.0 * third) * mf * wm2

    mu = jnp.mean(x, axis=1, keepdims=True)
    var = jnp.mean((x - mu) ** 2, axis=1, keepdims=True)
    xn = (x - mu) / jnp.sqrt(var + 1e-6)
    xn_ref[...] = xn.astype(jnp.bfloat16)

    running_ref[0:1, 0:s.shape[1]] = base + jnp.sum(s, axis=0, keepdims=True)


def _gather_body(s0_ref, s1_ref, xn_ref, cond_ref, gxn_ref, gcond_ref,
                 *, blk):
    i = pl.program_id(0)
    n = s0_ref.shape[1]
    srow = (jax.lax.broadcasted_iota(jnp.int32, (blk, n), 0).astype(jnp.float32)
            + jnp.float32(i * blk))
    s0b = jnp.broadcast_to(s0_ref[...].astype(jnp.float32), (blk, n))
    s1b = jnp.broadcast_to(s1_ref[...].astype(jnp.float32), (blk, n))
    p = (jnp.maximum(1.0 - jnp.abs(s0b - srow), 0.0)
         + jnp.maximum(1.0 - jnp.abs(s1b - srow), 0.0)).astype(jnp.bfloat16)
    gxn_ref[...] = jnp.dot(
        p, xn_ref[...], preferred_element_type=jnp.float32
    ).astype(jnp.bfloat16)
    gcond_ref[...] = jnp.dot(
        p, cond_ref[...], preferred_element_type=jnp.float32
    ).astype(jnp.bfloat16)


def _gmlp(xn, cond, wada_ref, bada_ref, w1_ref, w2_ref, ffn_chunks):
    t = cond.shape[0]
    ada = jnp.dot(cond, wada_ref[0], preferred_element_type=jnp.float32)
    ada = ada + bada_ref[0]
    d = ada.shape[1] // 3
    shift = ada[:, :d]
    scale = ada[:, d:2 * d]
    gate = ada[:, 2 * d:]
    h = xn.astype(jnp.float32) * (1.0 + scale) + shift
    hb = h.astype(jnp.bfloat16)
    w1 = w1_ref[0]
    w2 = w2_ref[0]
    ck = w1.shape[1] // ffn_chunks
    o = jnp.zeros((t, d), jnp.float32)
    for c in range(ffn_chunks):
        f = jnp.dot(hb, w1[:, c * ck:(c + 1) * ck],
                    preferred_element_type=jnp.float32)
        fb = jax.nn.gelu(f.astype(jnp.bfloat16))
        o = o + jnp.dot(fb, w2[c * ck:(c + 1) * ck, :],
                        preferred_element_type=jnp.float32)
    return gate * o


def _expert_ffn_body(gxn_ref, gcond_ref, wada_ref, bada_ref, w1_ref, w2_ref,
                     eo_ref, *, ffn_chunks):
    gh = _gmlp(gxn_ref[...], gcond_ref[...], wada_ref, bada_ref,
               w1_ref, w2_ref, ffn_chunks)
    eo_ref[...] = gh.astype(jnp.bfloat16)


def _shared_scatter_body(xn_ref, cond_ref, mask_ref, acc_ref,
                         s0_ref, w0_ref, s1_ref, w1_ref, eo_ref,
                         wada_ref, bada_ref, w1w_ref, w2w_ref, out_ref,
                         *, ffn_chunks, n_slots):
    gh = _gmlp(xn_ref[...], cond_ref[...], wada_ref, bada_ref,
               w1w_ref, w2w_ref, ffn_chunks)
    t = s0_ref.shape[0]
    siota = jax.lax.broadcasted_iota(jnp.int32, (t, n_slots), 1).astype(
        jnp.float32)
    hit0 = jnp.maximum(1.0 - jnp.abs(s0_ref[...].astype(jnp.float32) - siota),
                       0.0)
    hit1 = jnp.maximum(1.0 - jnp.abs(s1_ref[...].astype(jnp.float32) - siota),
                       0.0)
    pw = (hit0 * w0_ref[...] + hit1 * w1_ref[...]).astype(jnp.bfloat16)
    moe = jnp.dot(pw, eo_ref[...], preferred_element_type=jnp.float32)
    out_ref[...] = (acc_ref[...]
                    + (jnp.float32(1.0 / 3.0) * mask_ref[...]) * gh + moe)


def kernel(x, cond, mask, W_gate, Wada_sh, bada_sh, W1_sh, W2_sh,
           Wada_e, bada_e, W1_e, W2_e):
    b, n, d = x.shape
    nt = b * n
    dc = cond.shape[-1]
    n_exp = Wada_e.shape[0]
    cap = int(1.25 * 2 * nt / n_exp)
    n_slots = (n_exp - 1) * cap

    xf = x.reshape(nt, d)
    cf = cond.reshape(nt, dc).astype(jnp.bfloat16)
    mf = mask.reshape(nt, 1)
    wg = jnp.pad(W_gate, ((0, 0), (0, 1)))

    wada_all = jnp.concatenate(
        [Wada_sh[None], Wada_e[: n_exp - 1]], axis=0).astype(jnp.bfloat16)
    bada_all = jnp.concatenate(
        [bada_sh[None], bada_e[: n_exp - 1]], axis=0).reshape(n_exp, 1, 3 * d)
    w1_all = jnp.concatenate(
        [W1_sh[None], W1_e[: n_exp - 1]], axis=0).astype(jnp.bfloat16)
    w2_all = jnp.concatenate(
        [W2_sh[None], W2_e[: n_exp - 1]], axis=0).astype(jnp.bfloat16)
    ff = w1_all.shape[2]

    rt = 512
    acc0, xnb, s0, w0, s1, w1 = pl.pallas_call(
        functools.partial(_router_body, cap=cap, n_exp=n_exp),
        grid=(nt // rt,),
        in_specs=[
            pl.BlockSpec((rt, d), lambda i: (i, 0)),
            pl.BlockSpec((d, n_exp), lambda i: (0, 0)),
            pl.BlockSpec((rt, 1), lambda i: (i, 0)),
        ],
        out_specs=[
            pl.BlockSpec((rt, d), lambda i: (i, 0)),
            pl.BlockSpec((rt, d), lambda i: (i, 0)),
            pl.BlockSpec((rt, 1), lambda i: (i, 0)),
            pl.BlockSpec((rt, 1), lambda i: (i, 0)),
            pl.BlockSpec((rt, 1), lambda i: (i, 0)),
            pl.BlockSpec((rt, 1), lambda i: (i, 0)),
        ],
        out_shape=[
            jax.ShapeDtypeStruct((nt, d), jnp.float32),
            jax.ShapeDtypeStruct((nt, d), jnp.bfloat16),
            jax.ShapeDtypeStruct((nt, 1), jnp.int32),
            jax.ShapeDtypeStruct((nt, 1), jnp.float32),
            jax.ShapeDtypeStruct((nt, 1), jnp.int32),
            jax.ShapeDtypeStruct((nt, 1), jnp.float32),
        ],
        scratch_shapes=[pltpu.VMEM((8, 128), jnp.float32)],
    )(xf, wg, mf)

    s0r = s0.reshape(1, nt)
    s1r = s1.reshape(1, nt)

    gb = 256
    gxn, gcond = pl.pallas_call(
        functools.partial(_gather_body, blk=gb),
        grid=(n_slots // gb,),
        in_specs=[
            pl.BlockSpec((1, nt), lambda i: (0, 0)),
            pl.BlockSpec((1, nt), lambda i: (0, 0)),
            pl.BlockSpec((nt, d), lambda i: (0, 0)),
            pl.BlockSpec((nt, dc), lambda i: (0, 0)),
        ],
        out_specs=[
            pl.BlockSpec((gb, d), lambda i: (i, 0)),
            pl.BlockSpec((gb, dc), lambda i: (i, 0)),
        ],
        out_shape=[
            jax.ShapeDtypeStruct((n_slots, d), jnp.bfloat16),
            jax.ShapeDtypeStruct((n_slots, dc), jnp.bfloat16),
        ],
    )(s0r, s1r, xnb, cf)

    ft = 256
    spe = cap // ft
    eo = pl.pallas_call(
        functools.partial(_expert_ffn_body, ffn_chunks=4),
        grid=(n_exp - 1, spe),
        in_specs=[
            pl.BlockSpec((ft, d), lambda e, c: (e * spe + c, 0)),
            pl.BlockSpec((ft, dc), lambda e, c: (e * spe + c, 0)),
            pl.BlockSpec((1, dc, 3 * d), lambda e, c: (e + 1, 0, 0)),
            pl.BlockSpec((1, 1, 3 * d), lambda e, c: (e + 1, 0, 0)),
            pl.BlockSpec((1, d, ff), lambda e, c: (e + 1, 0, 0)),
            pl.BlockSpec((1, ff, d), lambda e, c: (e + 1, 0, 0)),
        ],
        out_specs=pl.BlockSpec((ft, d), lambda e, c: (e * spe + c, 0)),
        out_shape=jax.ShapeDtypeStruct((n_slots, d), jnp.bfloat16),
    )(gxn, gcond, wada_all, bada_all, w1_all, w2_all)

    out = pl.pallas_call(
        functools.partial(_shared_scatter_body, ffn_chunks=4,
                          n_slots=n_slots),
        grid=(nt // ft,),
        in_specs=[
            pl.BlockSpec((ft, d), lambda t: (t, 0)),
            pl.BlockSpec((ft, dc), lambda t: (t, 0)),
            pl.BlockSpec((ft, 1), lambda t: (t, 0)),
            pl.BlockSpec((ft, d), lambda t: (t, 0)),
            pl.BlockSpec((ft, 1), lambda t: (t, 0)),
            pl.BlockSpec((ft, 1), lambda t: (t, 0)),
            pl.BlockSpec((ft, 1), lambda t: (t, 0)),
            pl.BlockSpec((ft, 1), lambda t: (t, 0)),
            pl.BlockSpec((n_slots, d), lambda t: (0, 0)),
            pl.BlockSpec((1, dc, 3 * d), lambda t: (0, 0, 0)),
            pl.BlockSpec((1, 1, 3 * d), lambda t: (0, 0, 0)),
            pl.BlockSpec((1, d, ff), lambda t: (0, 0, 0)),
            pl.BlockSpec((1, ff, d), lambda t: (0, 0, 0)),
        ],
        out_specs=pl.BlockSpec((ft, d), lambda t: (t, 0)),
        out_shape=jax.ShapeDtypeStruct((nt, d), jnp.float32),
        input_output_aliases={3: 0},
    )(xnb, cf, mf, acc0, s0, w0, s1, w1, eo,
      wada_all, bada_all, w1_all, w2_all)

    return out.reshape(b, n, d)
